# Initial kernel scaffold; baseline (speedup 1.0000x reference)
import functools

import jax
import jax.numpy as jnp
from jax import lax
from jax.experimental import pallas as pl
from jax.experimental.pallas import tpu as pltpu
from jax.experimental.pallas import tpu_sc as plsc

NC = 2
NS = 16
LANES = 16
K = 128

_MESH = plsc.VectorSubcoreMesh(
    core_axis_name="c", subcore_axis_name="s", num_cores=NC, num_subcores=NS
)


def _deg_pass(dst4, npad):
    ch = dst4.shape[2]
    rps = npad // NS

    @functools.partial(
        pl.kernel,
        out_type=jax.ShapeDtypeStruct((NC, npad, LANES), jnp.float32),
        mesh=_MESH,
        scratch_types=[
            pltpu.VMEM((ch, K), jnp.int32),
            pltpu.VMEM((K, LANES), jnp.float32),
            pltpu.VMEM((rps, LANES), jnp.float32),
            pltpu.VMEM_SHARED((npad, LANES), jnp.float32),
        ],
    )
    def deg_k(dst_hbm, out_hbm, idxv, onesv, zbuf, acc):
        c = lax.axis_index("c")
        s = lax.axis_index("s")
        pltpu.sync_copy(dst_hbm.at[c, s], idxv)

        @pl.loop(0, K)
        def _ones(r):
            onesv[r, :] = jnp.ones((LANES,), jnp.float32)

        @pl.loop(0, rps)
        def _zeros(r):
            zbuf[r, :] = jnp.zeros((LANES,), jnp.float32)

        pltpu.sync_copy(zbuf, acc.at[pl.ds(s * rps, rps)])
        plsc.subcore_barrier()

        @pl.loop(0, ch)
        def _scat(i):
            pltpu.sync_copy(onesv, acc.at[idxv.at[i]], add=True)

        plsc.subcore_barrier()
        pltpu.sync_copy(acc.at[pl.ds(s * rps, rps)], out_hbm.at[c, pl.ds(s * rps, rps)])

    return deg_k(dst4)


def _edge_pass(table, src4, dst4, npad):
    ch = src4.shape[2]
    h = table.shape[1]
    rps = npad // NS
    zrows = rps // 4

    @functools.partial(
        pl.kernel,
        out_type=jax.ShapeDtypeStruct((NC, npad, h), jnp.float32),
        mesh=_MESH,
        scratch_types=[
            pltpu.VMEM((ch, K), jnp.int32),
            pltpu.VMEM((ch, K), jnp.int32),
            pltpu.VMEM((K, h), jnp.float32),
            pltpu.VMEM((zrows, h), jnp.float32),
            pltpu.VMEM_SHARED((npad, h), jnp.float32),
        ],
    )
    def edge_k(table_hbm, src_hbm, dst_hbm, out_hbm, srcv, dstv, gbuf, zbuf, acc):
        c = lax.axis_index("c")
        s = lax.axis_index("s")
        pltpu.sync_copy(src_hbm.at[c, s], srcv)
        pltpu.sync_copy(dst_hbm.at[c, s], dstv)

        @pl.loop(0, zrows)
        def _zr(r):
            @pl.loop(0, h // LANES)
            def _zg(g):
                zbuf[r, pl.ds(g * LANES, LANES)] = jnp.zeros((LANES,), jnp.float32)

        for q in range(4):
            pltpu.sync_copy(zbuf, acc.at[pl.ds(s * rps + q * zrows, zrows)])
        plsc.subcore_barrier()

        @pl.loop(0, ch)
        def _go(i):
            pltpu.sync_copy(table_hbm.at[srcv.at[i]], gbuf)
            pltpu.sync_copy(gbuf, acc.at[dstv.at[i]], add=True)

        plsc.subcore_barrier()
        pltpu.sync_copy(acc.at[pl.ds(s * rps, rps)], out_hbm.at[c, pl.ds(s * rps, rps)])

    return edge_k(table, src4, dst4)


def _dinv_of(deg_ref, n):
    cnt = deg_ref[0, :n, :1] + deg_ref[1, :n, :1] + 1.0
    return lax.rsqrt(cnt)


def _mm_scale(x, w, deg):
    n = x.shape[0]

    def body(x_ref, w_ref, deg_ref, o_ref):
        dinv = _dinv_of(deg_ref, n)
        hmat = jnp.dot(
            x_ref[...], w_ref[...],
            preferred_element_type=jnp.float32, precision=lax.Precision.HIGHEST,
        )
        o_ref[...] = hmat * dinv

    return pl.pallas_call(
        body, out_shape=jax.ShapeDtypeStruct((n, w.shape[1]), jnp.float32)
    )(x, w, deg)


def _mid(accp, hp, deg, b, g, bt, w2):
    n = hp.shape[0]

    def body(acc_ref, h_ref, deg_ref, b_ref, g_ref, bt_ref, w2_ref, o_ref):
        dinv = _dinv_of(deg_ref, n)
        t = (acc_ref[0, :n, :] + acc_ref[1, :n, :] + h_ref[...]) * dinv + b_ref[...]
        mu = jnp.mean(t, axis=0, keepdims=True)
        xc = t - mu
        var = jnp.mean(xc * xc, axis=0, keepdims=True)
        z = xc * lax.rsqrt(var + 1e-5) * g_ref[...] + bt_ref[...]
        z = jnp.maximum(z, 0.0)
        o_ref[...] = jnp.dot(
            z, w2_ref[...],
            preferred_element_type=jnp.float32, precision=lax.Precision.HIGHEST,
        ) * dinv

    return pl.pallas_call(
        body, out_shape=jax.ShapeDtypeStruct((n, w2.shape[1]), jnp.float32)
    )(accp, hp, deg, b, g, bt, w2)


def _fin(accp, hp, deg, b, g, bt, wc, bc):
    n = hp.shape[0]

    def body(acc_ref, h_ref, deg_ref, b_ref, g_ref, bt_ref, wc_ref, bc_ref, o_ref):
        dinv = _dinv_of(deg_ref, n)
        t = (acc_ref[0, :n, :] + acc_ref[1, :n, :] + h_ref[...]) * dinv + b_ref[...]
        mu = jnp.mean(t, axis=0, keepdims=True)
        xc = t - mu
        var = jnp.mean(xc * xc, axis=0, keepdims=True)
        z = xc * lax.rsqrt(var + 1e-5) * g_ref[...] + bt_ref[...]
        z = jnp.maximum(z, 0.0)
        o_ref[...] = jnp.dot(
            z, wc_ref[...],
            preferred_element_type=jnp.float32, precision=lax.Precision.HIGHEST,
        ) + bc_ref[...]

    return pl.pallas_call(
        body, out_shape=jax.ShapeDtypeStruct((n, wc.shape[1]), jnp.float32)
    )(accp, hp, deg, b, g, bt, wc, bc)


def kernel(x, edge_index, W1, b1, g1, bt1, W2, b2, g2, bt2, Wc, bc):
    n = x.shape[0]
    e = edge_index.shape[1]
    npad = -(-(n + 1) // (NS * 8)) * (NS * 8)
    ch = -(-e // (NC * NS * K))
    pad = NC * NS * ch * K - e
    src = jnp.concatenate([edge_index[0], jnp.zeros((pad,), edge_index.dtype)])
    dst = jnp.concatenate([edge_index[1], jnp.full((pad,), n, edge_index.dtype)])
    src4 = src.reshape(NC, NS, ch, K)
    dst4 = dst.reshape(NC, NS, ch, K)

    deg = _deg_pass(dst4, npad)
    h1 = _mm_scale(x, W1, deg)
    acc1 = _edge_pass(h1, src4, dst4, npad)
    h2 = _mid(
        acc1, h1, deg,
        b1.reshape(1, -1), g1.reshape(1, -1), bt1.reshape(1, -1), W2,
    )
    acc2 = _edge_pass(h2, src4, dst4, npad)
    return _fin(
        acc2, h2, deg,
        b2.reshape(1, -1), g2.reshape(1, -1), bt2.reshape(1, -1), Wc,
        bc.reshape(1, -1),
    )


# speedup vs baseline: 12.6246x; 12.6246x over previous
import functools

import jax
import jax.numpy as jnp
from jax import lax
from jax.experimental import pallas as pl
from jax.experimental.pallas import tpu as pltpu
from jax.experimental.pallas import tpu_sc as plsc

NC = 2
NS = 16
LANES = 16
K = 128

_MESH = plsc.VectorSubcoreMesh(
    core_axis_name="c", subcore_axis_name="s", num_cores=NC, num_subcores=NS
)


def _fill(gbuf, rows, val):
    @pl.loop(0, rows)
    def _r(r):
        @pl.loop(0, 128 // LANES)
        def _g(g):
            gbuf[r, pl.ds(g * LANES, LANES)] = jnp.full((LANES,), val, jnp.float32)


def _zero_acc(gbuf, acc, s, rps):
    off = 0
    while off < rps:
        step = min(K, rps - off)
        pltpu.sync_copy(gbuf.at[pl.ds(0, step)], acc.at[pl.ds(s * rps + off, step)])
        off += step


def _acc_to_hbm(acc, gbuf, out_hbm, c, s, rps):
    off = 0
    while off < rps:
        step = min(K, rps - off)
        pltpu.sync_copy(acc.at[pl.ds(s * rps + off, step)], gbuf.at[pl.ds(0, step)])
        pltpu.sync_copy(gbuf.at[pl.ds(0, step)], out_hbm.at[c, pl.ds(s * rps + off, step)])
        off += step


def _deg_pass(dst4, npad):
    ch = dst4.shape[2]
    rps = npad // NS

    @functools.partial(
        pl.kernel,
        out_type=jax.ShapeDtypeStruct((NC, npad, 128), jnp.float32),
        mesh=_MESH,
        scratch_types=[
            pltpu.VMEM((ch, K), jnp.int32),
            pltpu.VMEM((K, 128), jnp.float32),
            pltpu.VMEM_SHARED((npad, 128), jnp.float32),
        ],
    )
    def deg_k(dst_hbm, out_hbm, dstv, gbuf, acc):
        c = lax.axis_index("c")
        s = lax.axis_index("s")
        pltpu.sync_copy(dst_hbm.at[c, s], dstv)
        _fill(gbuf, K, 0.0)
        _zero_acc(gbuf, acc, s, rps)
        _fill(gbuf, K, 1.0)
        plsc.subcore_barrier()

        @pl.loop(0, ch)
        def _scat(i):
            pltpu.sync_copy(gbuf, acc.at[dstv.at[i]], add=True)

        plsc.subcore_barrier()
        _acc_to_hbm(acc, gbuf, out_hbm, c, s, rps)

    return deg_k(dst4)


def _edge_pass(table, src4, dst4, npad):
    ch = src4.shape[2]
    h = table.shape[1]
    rps = npad // NS

    @functools.partial(
        pl.kernel,
        out_type=jax.ShapeDtypeStruct((NC, npad, h), jnp.float32),
        mesh=_MESH,
        scratch_types=[
            pltpu.VMEM((ch, K), jnp.int32),
            pltpu.VMEM((ch, K), jnp.int32),
            pltpu.VMEM((K, h), jnp.float32),
            pltpu.VMEM_SHARED((npad, h), jnp.float32),
        ],
    )
    def edge_k(table_hbm, src_hbm, dst_hbm, out_hbm, srcv, dstv, gbuf, acc):
        c = lax.axis_index("c")
        s = lax.axis_index("s")
        pltpu.sync_copy(src_hbm.at[c, s], srcv)
        pltpu.sync_copy(dst_hbm.at[c, s], dstv)
        _fill(gbuf, K, 0.0)
        _zero_acc(gbuf, acc, s, rps)
        plsc.subcore_barrier()

        @pl.loop(0, ch)
        def _go(i):
            pltpu.sync_copy(table_hbm.at[srcv.at[i]], gbuf)
            pltpu.sync_copy(gbuf, acc.at[dstv.at[i]], add=True)

        plsc.subcore_barrier()
        _acc_to_hbm(acc, gbuf, out_hbm, c, s, rps)

    return edge_k(table, src4, dst4)


def _dinv_of(deg_ref, n):
    cnt = deg_ref[0, :n, :1] + deg_ref[1, :n, :1] + 1.0
    return lax.rsqrt(cnt)


def _mm_scale(x, w, deg):
    n = x.shape[0]

    def body(x_ref, w_ref, deg_ref, o_ref):
        dinv = _dinv_of(deg_ref, n)
        hmat = jnp.dot(
            x_ref[...], w_ref[...],
            preferred_element_type=jnp.float32, precision=lax.Precision.HIGHEST,
        )
        o_ref[...] = hmat * dinv

    return pl.pallas_call(
        body, out_shape=jax.ShapeDtypeStruct((n, w.shape[1]), jnp.float32)
    )(x, w, deg)


def _mid(accp, hp, deg, b, g, bt, w2):
    n = hp.shape[0]

    def body(acc_ref, h_ref, deg_ref, b_ref, g_ref, bt_ref, w2_ref, o_ref):
        dinv = _dinv_of(deg_ref, n)
        t = (acc_ref[0, :n, :] + acc_ref[1, :n, :] + h_ref[...]) * dinv + b_ref[...]
        mu = jnp.mean(t, axis=0, keepdims=True)
        xc = t - mu
        var = jnp.mean(xc * xc, axis=0, keepdims=True)
        z = xc * lax.rsqrt(var + 1e-5) * g_ref[...] + bt_ref[...]
        z = jnp.maximum(z, 0.0)
        o_ref[...] = jnp.dot(
            z, w2_ref[...],
            preferred_element_type=jnp.float32, precision=lax.Precision.HIGHEST,
        ) * dinv

    return pl.pallas_call(
        body, out_shape=jax.ShapeDtypeStruct((n, w2.shape[1]), jnp.float32)
    )(accp, hp, deg, b, g, bt, w2)


def _fin(accp, hp, deg, b, g, bt, wc, bc):
    n = hp.shape[0]

    def body(acc_ref, h_ref, deg_ref, b_ref, g_ref, bt_ref, wc_ref, bc_ref, o_ref):
        dinv = _dinv_of(deg_ref, n)
        t = (acc_ref[0, :n, :] + acc_ref[1, :n, :] + h_ref[...]) * dinv + b_ref[...]
        mu = jnp.mean(t, axis=0, keepdims=True)
        xc = t - mu
        var = jnp.mean(xc * xc, axis=0, keepdims=True)
        z = xc * lax.rsqrt(var + 1e-5) * g_ref[...] + bt_ref[...]
        z = jnp.maximum(z, 0.0)
        o_ref[...] = jnp.dot(
            z, wc_ref[...],
            preferred_element_type=jnp.float32, precision=lax.Precision.HIGHEST,
        ) + bc_ref[...]

    return pl.pallas_call(
        body, out_shape=jax.ShapeDtypeStruct((n, wc.shape[1]), jnp.float32)
    )(accp, hp, deg, b, g, bt, wc, bc)


def kernel(x, edge_index, W1, b1, g1, bt1, W2, b2, g2, bt2, Wc, bc):
    n = x.shape[0]
    e = edge_index.shape[1]
    npad = -(-(n + 1) // (NS * 8)) * (NS * 8)
    ch = -(-e // (NC * NS * K))
    pad = NC * NS * ch * K - e
    src = jnp.concatenate([edge_index[0], jnp.zeros((pad,), edge_index.dtype)])
    dst = jnp.concatenate([edge_index[1], jnp.full((pad,), n, edge_index.dtype)])
    src4 = src.reshape(NC, NS, ch, K)
    dst4 = dst.reshape(NC, NS, ch, K)

    deg = _deg_pass(dst4, npad)
    h1 = _mm_scale(x, W1, deg)
    acc1 = _edge_pass(h1, src4, dst4, npad)
    h2 = _mid(
        acc1, h1, deg,
        b1.reshape(1, -1), g1.reshape(1, -1), bt1.reshape(1, -1), W2,
    )
    acc2 = _edge_pass(h2, src4, dst4, npad)
    return _fin(
        acc2, h2, deg,
        b2.reshape(1, -1), g2.reshape(1, -1), bt2.reshape(1, -1), Wc,
        bc.reshape(1, -1),
    )


# speedup vs baseline: 24.5366x; 1.9436x over previous
import functools

import jax
import jax.numpy as jnp
from jax import lax
from jax.experimental import pallas as pl
from jax.experimental.pallas import tpu as pltpu
from jax.experimental.pallas import tpu_sc as plsc

NC = 2
NS = 16
LANES = 16
K = 128

_MESH = plsc.VectorSubcoreMesh(
    core_axis_name="c", subcore_axis_name="s", num_cores=NC, num_subcores=NS
)


def _fill(gbuf, rows, val):
    @pl.loop(0, rows)
    def _r(r):
        @pl.loop(0, 128 // LANES)
        def _g(g):
            gbuf[r, pl.ds(g * LANES, LANES)] = jnp.full((LANES,), val, jnp.float32)


def _zero_acc(gbuf, acc, s, rps):
    off = 0
    while off < rps:
        step = min(K, rps - off)
        pltpu.sync_copy(gbuf.at[pl.ds(0, step)], acc.at[pl.ds(s * rps + off, step)])
        off += step


def _acc_to_hbm(acc, gbuf, out_hbm, c, s, rps):
    off = 0
    while off < rps:
        step = min(K, rps - off)
        pltpu.sync_copy(acc.at[pl.ds(s * rps + off, step)], gbuf.at[pl.ds(0, step)])
        pltpu.sync_copy(gbuf.at[pl.ds(0, step)], out_hbm.at[c, pl.ds(s * rps + off, step)])
        off += step


_SHIFT = 14
_MASK = (1 << _SHIFT) - 1


def _unpack(pslab, i, sbuf, dbuf):
    for g in range(K // LANES):
        p = pslab[i, pl.ds(g * LANES, LANES)]
        if sbuf is not None:
            sbuf[pl.ds(g * LANES, LANES)] = p & _MASK
        dbuf[pl.ds(g * LANES, LANES)] = lax.shift_right_logical(p, _SHIFT)


def _deg_pass(packed4, npad):
    ch = packed4.shape[2]
    rps = npad // NS

    @functools.partial(
        pl.kernel,
        out_type=jax.ShapeDtypeStruct((NC, npad, 128), jnp.float32),
        mesh=_MESH,
        scratch_types=[
            pltpu.VMEM((ch, K), jnp.int32),
            pltpu.VMEM((K, 128), jnp.float32),
            pltpu.VMEM((K,), jnp.int32),
            pltpu.VMEM((K,), jnp.int32),
            pltpu.VMEM_SHARED((npad, 128), jnp.float32),
            pltpu.SemaphoreType.DMA,
            pltpu.SemaphoreType.DMA,
        ],
    )
    def deg_k(p_hbm, out_hbm, pslab, gbuf, da, db, acc, ssa, ssb):
        c = lax.axis_index("c")
        s = lax.axis_index("s")
        pltpu.sync_copy(p_hbm.at[c, s], pslab)
        _fill(gbuf, K, 0.0)
        _zero_acc(gbuf, acc, s, rps)
        _fill(gbuf, K, 1.0)
        plsc.subcore_barrier()

        @pl.loop(0, ch, step=2)
        def _scat(j0):
            for p, (dbuf, sem) in enumerate(((da, ssa), (db, ssb))):
                i = j0 + p
                @pl.when(i >= 2)
                def _w():
                    pltpu.make_async_copy(gbuf, acc.at[dbuf], sem).wait()
                _unpack(pslab, i, None, dbuf)
                pltpu.async_copy(gbuf, acc.at[dbuf], sem, add=True)

        pltpu.make_async_copy(gbuf, acc.at[da], ssa).wait()
        pltpu.make_async_copy(gbuf, acc.at[db], ssb).wait()
        plsc.subcore_barrier()
        _acc_to_hbm(acc, gbuf, out_hbm, c, s, rps)

    return deg_k(packed4)


def _edge_pass(table, packed4, npad):
    ch = packed4.shape[2]
    h = table.shape[1]
    rps = npad // NS

    @functools.partial(
        pl.kernel,
        out_type=jax.ShapeDtypeStruct((NC, npad, h), jnp.float32),
        mesh=_MESH,
        scratch_types=[
            pltpu.VMEM((ch, K), jnp.int32),
            pltpu.VMEM((K, h), jnp.float32),
            pltpu.VMEM((K, h), jnp.float32),
            pltpu.VMEM((K,), jnp.int32),
            pltpu.VMEM((K,), jnp.int32),
            pltpu.VMEM((K,), jnp.int32),
            pltpu.VMEM((K,), jnp.int32),
            pltpu.VMEM_SHARED((npad, h), jnp.float32),
            pltpu.SemaphoreType.DMA,
            pltpu.SemaphoreType.DMA,
        ],
    )
    def edge_k(table_hbm, p_hbm, out_hbm, pslab, gba, gbb, sa, sb, da, db,
               acc, gsa, gsb):
        c = lax.axis_index("c")
        s = lax.axis_index("s")
        pltpu.sync_copy(p_hbm.at[c, s], pslab)
        _fill(gba, K, 0.0)
        _zero_acc(gba, acc, s, rps)
        plsc.subcore_barrier()

        _unpack(pslab, 0, sa, da)
        pltpu.async_copy(table_hbm.at[sa], gba, gsa)

        @pl.loop(0, ch, step=2)
        def _go(j0):
            _unpack(pslab, j0 + 1, sb, db)
            pltpu.async_copy(table_hbm.at[sb], gbb, gsb)
            pltpu.make_async_copy(table_hbm.at[sa], gba, gsa).wait()
            pltpu.sync_copy(gba, acc.at[da], add=True)
            @pl.when(j0 + 2 < ch)
            def _next():
                _unpack(pslab, j0 + 2, sa, da)
                pltpu.async_copy(table_hbm.at[sa], gba, gsa)
            pltpu.make_async_copy(table_hbm.at[sb], gbb, gsb).wait()
            pltpu.sync_copy(gbb, acc.at[db], add=True)

        plsc.subcore_barrier()
        _acc_to_hbm(acc, gba, out_hbm, c, s, rps)

    return edge_k(table, packed4)


def _dinv_of(deg_ref, n):
    cnt = deg_ref[0, :n, :1] + deg_ref[1, :n, :1] + 1.0
    return lax.rsqrt(cnt)


def _mm_scale(x, w, deg):
    n = x.shape[0]

    def body(x_ref, w_ref, deg_ref, o_ref):
        dinv = _dinv_of(deg_ref, n)
        hmat = jnp.dot(
            x_ref[...], w_ref[...],
            preferred_element_type=jnp.float32, precision=lax.Precision.HIGHEST,
        )
        o_ref[...] = hmat * dinv

    return pl.pallas_call(
        body, out_shape=jax.ShapeDtypeStruct((n, w.shape[1]), jnp.float32)
    )(x, w, deg)


def _mid(accp, hp, deg, b, g, bt, w2):
    n = hp.shape[0]

    def body(acc_ref, h_ref, deg_ref, b_ref, g_ref, bt_ref, w2_ref, o_ref):
        dinv = _dinv_of(deg_ref, n)
        t = (acc_ref[0, :n, :] + acc_ref[1, :n, :] + h_ref[...]) * dinv + b_ref[...]
        mu = jnp.mean(t, axis=0, keepdims=True)
        xc = t - mu
        var = jnp.mean(xc * xc, axis=0, keepdims=True)
        z = xc * lax.rsqrt(var + 1e-5) * g_ref[...] + bt_ref[...]
        z = jnp.maximum(z, 0.0)
        o_ref[...] = jnp.dot(
            z, w2_ref[...],
            preferred_element_type=jnp.float32, precision=lax.Precision.HIGHEST,
        ) * dinv

    return pl.pallas_call(
        body, out_shape=jax.ShapeDtypeStruct((n, w2.shape[1]), jnp.float32)
    )(accp, hp, deg, b, g, bt, w2)


def _fin(accp, hp, deg, b, g, bt, wc, bc):
    n = hp.shape[0]

    def body(acc_ref, h_ref, deg_ref, b_ref, g_ref, bt_ref, wc_ref, bc_ref, o_ref):
        dinv = _dinv_of(deg_ref, n)
        t = (acc_ref[0, :n, :] + acc_ref[1, :n, :] + h_ref[...]) * dinv + b_ref[...]
        mu = jnp.mean(t, axis=0, keepdims=True)
        xc = t - mu
        var = jnp.mean(xc * xc, axis=0, keepdims=True)
        z = xc * lax.rsqrt(var + 1e-5) * g_ref[...] + bt_ref[...]
        z = jnp.maximum(z, 0.0)
        o_ref[...] = jnp.dot(
            z, wc_ref[...],
            preferred_element_type=jnp.float32, precision=lax.Precision.HIGHEST,
        ) + bc_ref[...]

    return pl.pallas_call(
        body, out_shape=jax.ShapeDtypeStruct((n, wc.shape[1]), jnp.float32)
    )(accp, hp, deg, b, g, bt, wc, bc)


def kernel(x, edge_index, W1, b1, g1, bt1, W2, b2, g2, bt2, Wc, bc):
    n = x.shape[0]
    e = edge_index.shape[1]
    npad = -(-(n + 1) // (NS * 8)) * (NS * 8)
    assert npad <= (1 << (31 - _SHIFT)) and n <= _MASK
    ch = -(-e // (NC * NS * K))
    ch += ch % 2
    pad = NC * NS * ch * K - e
    ar = jnp.arange(pad, dtype=edge_index.dtype)
    src = jnp.concatenate([edge_index[0], ar % 8])
    dst = jnp.concatenate([edge_index[1], n + ar % (npad - n)])
    packed4 = (src | (dst << _SHIFT)).reshape(NC, NS, ch, K)

    deg = _deg_pass(packed4, npad)
    h1 = _mm_scale(x, W1, deg)
    acc1 = _edge_pass(h1, packed4, npad)
    h2 = _mid(
        acc1, h1, deg,
        b1.reshape(1, -1), g1.reshape(1, -1), bt1.reshape(1, -1), W2,
    )
    acc2 = _edge_pass(h2, packed4, npad)
    return _fin(
        acc2, h2, deg,
        b2.reshape(1, -1), g2.reshape(1, -1), bt2.reshape(1, -1), Wc,
        bc.reshape(1, -1),
    )


# speedup vs baseline: 26.2761x; 1.0709x over previous
import dataclasses
import functools

import jax
import jax.numpy as jnp
from jax import lax
from jax.experimental import pallas as pl
from jax.experimental.pallas import tpu as pltpu
from jax.experimental.pallas import tpu_sc as plsc

NC = 2
NS = 16
LANES = 16
K = 128

_MESH = plsc.VectorSubcoreMesh(
    core_axis_name="c", subcore_axis_name="s", num_cores=NC, num_subcores=NS
)

_CP_NO_LAYOUT = pltpu.CompilerParams()
if "needs_layout_passes" in pltpu.CompilerParams.__dataclass_fields__:
    _CP_NO_LAYOUT = dataclasses.replace(_CP_NO_LAYOUT, needs_layout_passes=False)


def _fill(gbuf, rows, val):
    @pl.loop(0, rows)
    def _r(r):
        @pl.loop(0, 128 // LANES)
        def _g(g):
            gbuf[r, pl.ds(g * LANES, LANES)] = jnp.full((LANES,), val, jnp.float32)


def _zero_acc(gbuf, acc, s, rps):
    off = 0
    while off < rps:
        step = min(K, rps - off)
        pltpu.sync_copy(gbuf.at[pl.ds(0, step)], acc.at[pl.ds(s * rps + off, step)])
        off += step


def _acc_to_hbm(acc, gbuf, out_hbm, c, s, rps):
    off = 0
    while off < rps:
        step = min(K, rps - off)
        pltpu.sync_copy(acc.at[pl.ds(s * rps + off, step)], gbuf.at[pl.ds(0, step)])
        pltpu.sync_copy(gbuf.at[pl.ds(0, step)], out_hbm.at[c, pl.ds(s * rps + off, step)])
        off += step


_SHIFT = 14
_MASK = (1 << _SHIFT) - 1


def _unpack(pslab, i, sbuf, dbuf):
    for g in range(K // LANES):
        p = pslab[i, pl.ds(g * LANES, LANES)]
        if sbuf is not None:
            sbuf[pl.ds(g * LANES, LANES)] = p & _MASK
        dbuf[pl.ds(g * LANES, LANES)] = lax.shift_right_logical(p, _SHIFT)


def _deg_pass(packed4, npad):
    ch = packed4.shape[2]
    flat = packed4.reshape(NC, NS, ch * K)

    @functools.partial(
        pl.kernel,
        out_type=jax.ShapeDtypeStruct((NC, NS, npad), jnp.float32),
        mesh=_MESH,
        compiler_params=_CP_NO_LAYOUT,
        scratch_types=[
            pltpu.VMEM((ch * K,), jnp.int32),
            pltpu.VMEM((npad,), jnp.float32),
        ],
    )
    def deg_k(p_hbm, out_hbm, pslab, hist):
        c = lax.axis_index("c")
        s = lax.axis_index("s")
        pltpu.sync_copy(p_hbm.at[c, s], pslab)

        @pl.loop(0, npad // LANES)
        def _z(r):
            hist[pl.ds(r * LANES, LANES)] = jnp.zeros((LANES,), jnp.float32)

        ones = jnp.ones((LANES,), jnp.float32)

        @pl.loop(0, ch * K // LANES)
        def _h(i):
            d = lax.shift_right_logical(pslab[pl.ds(i * LANES, LANES)], _SHIFT)
            plsc.addupdate_scatter(hist, [d], ones)

        pltpu.sync_copy(hist, out_hbm.at[c, s])

    return deg_k(flat)


def _edge_pass(table, packed4, npad):
    ch = packed4.shape[2]
    h = table.shape[1]
    rps = npad // NS

    @functools.partial(
        pl.kernel,
        out_type=jax.ShapeDtypeStruct((NC, npad, h), jnp.float32),
        mesh=_MESH,
        scratch_types=[
            pltpu.VMEM((ch, K), jnp.int32),
            pltpu.VMEM((K, h), jnp.float32),
            pltpu.VMEM((K, h), jnp.float32),
            pltpu.VMEM((K,), jnp.int32),
            pltpu.VMEM((K,), jnp.int32),
            pltpu.VMEM((K,), jnp.int32),
            pltpu.VMEM((K,), jnp.int32),
            pltpu.VMEM_SHARED((npad, h), jnp.float32),
            pltpu.SemaphoreType.DMA,
            pltpu.SemaphoreType.DMA,
        ],
    )
    def edge_k(table_hbm, p_hbm, out_hbm, pslab, gba, gbb, sa, sb, da, db,
               acc, gsa, gsb):
        c = lax.axis_index("c")
        s = lax.axis_index("s")
        pltpu.sync_copy(p_hbm.at[c, s], pslab)
        _fill(gba, K, 0.0)
        _zero_acc(gba, acc, s, rps)
        plsc.subcore_barrier()

        _unpack(pslab, 0, sa, da)
        pltpu.async_copy(table_hbm.at[sa], gba, gsa)

        @pl.loop(0, ch, step=2)
        def _go(j0):
            _unpack(pslab, j0 + 1, sb, db)
            pltpu.async_copy(table_hbm.at[sb], gbb, gsb)
            pltpu.make_async_copy(table_hbm.at[sa], gba, gsa).wait()
            pltpu.sync_copy(gba, acc.at[da], add=True)
            @pl.when(j0 + 2 < ch)
            def _next():
                _unpack(pslab, j0 + 2, sa, da)
                pltpu.async_copy(table_hbm.at[sa], gba, gsa)
            pltpu.make_async_copy(table_hbm.at[sb], gbb, gsb).wait()
            pltpu.sync_copy(gbb, acc.at[db], add=True)

        plsc.subcore_barrier()
        _acc_to_hbm(acc, gba, out_hbm, c, s, rps)

    return edge_k(table, packed4)


def _dinv_of(deg_ref, n):
    nw = deg_ref.shape[0]
    cnt = lax.dot_general(
        deg_ref[...], jnp.ones((nw, 1), jnp.float32),
        (((0,), (0,)), ((), ())),
        preferred_element_type=jnp.float32, precision=lax.Precision.HIGHEST,
    )
    return lax.rsqrt(cnt[:n] + 1.0)


def _mm_scale(x, w, deg):
    n = x.shape[0]

    def body(x_ref, w_ref, deg_ref, o_ref):
        dinv = _dinv_of(deg_ref, n)
        hmat = jnp.dot(
            x_ref[...], w_ref[...],
            preferred_element_type=jnp.float32, precision=lax.Precision.HIGHEST,
        )
        o_ref[...] = hmat * dinv

    return pl.pallas_call(
        body, out_shape=jax.ShapeDtypeStruct((n, w.shape[1]), jnp.float32)
    )(x, w, deg)


def _mid(accp, hp, deg, b, g, bt, w2):
    n = hp.shape[0]

    def body(acc_ref, h_ref, deg_ref, b_ref, g_ref, bt_ref, w2_ref, o_ref):
        dinv = _dinv_of(deg_ref, n)
        t = (acc_ref[0, :n, :] + acc_ref[1, :n, :] + h_ref[...]) * dinv + b_ref[...]
        mu = jnp.mean(t, axis=0, keepdims=True)
        xc = t - mu
        var = jnp.mean(xc * xc, axis=0, keepdims=True)
        z = xc * lax.rsqrt(var + 1e-5) * g_ref[...] + bt_ref[...]
        z = jnp.maximum(z, 0.0)
        o_ref[...] = jnp.dot(
            z, w2_ref[...],
            preferred_element_type=jnp.float32, precision=lax.Precision.HIGHEST,
        ) * dinv

    return pl.pallas_call(
        body, out_shape=jax.ShapeDtypeStruct((n, w2.shape[1]), jnp.float32)
    )(accp, hp, deg, b, g, bt, w2)


def _fin(accp, hp, deg, b, g, bt, wc, bc):
    n = hp.shape[0]

    def body(acc_ref, h_ref, deg_ref, b_ref, g_ref, bt_ref, wc_ref, bc_ref, o_ref):
        dinv = _dinv_of(deg_ref, n)
        t = (acc_ref[0, :n, :] + acc_ref[1, :n, :] + h_ref[...]) * dinv + b_ref[...]
        mu = jnp.mean(t, axis=0, keepdims=True)
        xc = t - mu
        var = jnp.mean(xc * xc, axis=0, keepdims=True)
        z = xc * lax.rsqrt(var + 1e-5) * g_ref[...] + bt_ref[...]
        z = jnp.maximum(z, 0.0)
        o_ref[...] = jnp.dot(
            z, wc_ref[...],
            preferred_element_type=jnp.float32, precision=lax.Precision.HIGHEST,
        ) + bc_ref[...]

    return pl.pallas_call(
        body, out_shape=jax.ShapeDtypeStruct((n, wc.shape[1]), jnp.float32)
    )(accp, hp, deg, b, g, bt, wc, bc)


def kernel(x, edge_index, W1, b1, g1, bt1, W2, b2, g2, bt2, Wc, bc):
    n = x.shape[0]
    e = edge_index.shape[1]
    npad = -(-(n + 1) // (NS * 8)) * (NS * 8)
    assert npad <= (1 << (31 - _SHIFT)) and n <= _MASK
    ch = -(-e // (NC * NS * K))
    ch += ch % 2
    pad = NC * NS * ch * K - e
    ar = jnp.arange(pad, dtype=edge_index.dtype)
    src = jnp.concatenate([edge_index[0], ar % 8])
    dst = jnp.concatenate([edge_index[1], n + ar % (npad - n)])
    packed4 = (src | (dst << _SHIFT)).reshape(NC, NS, ch, K)

    deg = _deg_pass(packed4, npad).reshape(NC * NS, npad)
    h1 = _mm_scale(x, W1, deg)
    acc1 = _edge_pass(h1, packed4, npad)
    h2 = _mid(
        acc1, h1, deg,
        b1.reshape(1, -1), g1.reshape(1, -1), bt1.reshape(1, -1), W2,
    )
    acc2 = _edge_pass(h2, packed4, npad)
    return _fin(
        acc2, h2, deg,
        b2.reshape(1, -1), g2.reshape(1, -1), bt2.reshape(1, -1), Wc,
        bc.reshape(1, -1),
    )


# speedup vs baseline: 26.7647x; 1.0186x over previous
import dataclasses
import functools

import jax
import jax.numpy as jnp
from jax import lax
from jax.experimental import pallas as pl
from jax.experimental.pallas import tpu as pltpu
from jax.experimental.pallas import tpu_sc as plsc

NC = 2
NS = 16
LANES = 16
K = 128

_MESH = plsc.VectorSubcoreMesh(
    core_axis_name="c", subcore_axis_name="s", num_cores=NC, num_subcores=NS
)

_CP_NO_LAYOUT = pltpu.CompilerParams()
if "needs_layout_passes" in pltpu.CompilerParams.__dataclass_fields__:
    _CP_NO_LAYOUT = dataclasses.replace(_CP_NO_LAYOUT, needs_layout_passes=False)


def _fill(gbuf, rows, val):
    @pl.loop(0, rows)
    def _r(r):
        @pl.loop(0, 128 // LANES)
        def _g(g):
            gbuf[r, pl.ds(g * LANES, LANES)] = jnp.full((LANES,), val, jnp.float32)


def _zero_acc(gbuf, acc, s, rps):
    off = 0
    while off < rps:
        step = min(K, rps - off)
        pltpu.sync_copy(gbuf.at[pl.ds(0, step)], acc.at[pl.ds(s * rps + off, step)])
        off += step


def _acc_to_hbm(acc, gbuf, out_hbm, c, s, rps):
    off = 0
    while off < rps:
        step = min(K, rps - off)
        pltpu.sync_copy(acc.at[pl.ds(s * rps + off, step)], gbuf.at[pl.ds(0, step)])
        pltpu.sync_copy(gbuf.at[pl.ds(0, step)], out_hbm.at[c, pl.ds(s * rps + off, step)])
        off += step


_SHIFT = 14
_MASK = (1 << _SHIFT) - 1


def _unpack(pslab, i, sbuf, dbuf):
    for g in range(K // LANES):
        p = pslab[i, pl.ds(g * LANES, LANES)]
        if sbuf is not None:
            sbuf[pl.ds(g * LANES, LANES)] = p & _MASK
        dbuf[pl.ds(g * LANES, LANES)] = lax.shift_right_logical(p, _SHIFT)


def _deg_pass(packed4, npad):
    ch = packed4.shape[2]
    flat = packed4.reshape(NC, NS, ch * K)

    @functools.partial(
        pl.kernel,
        out_type=jax.ShapeDtypeStruct((NC, NS, npad), jnp.float32),
        mesh=_MESH,
        compiler_params=_CP_NO_LAYOUT,
        scratch_types=[
            pltpu.VMEM((ch * K,), jnp.int32),
            pltpu.VMEM((npad,), jnp.float32),
        ],
    )
    def deg_k(p_hbm, out_hbm, pslab, hist):
        c = lax.axis_index("c")
        s = lax.axis_index("s")
        pltpu.sync_copy(p_hbm.at[c, s], pslab)

        @pl.loop(0, npad // LANES)
        def _z(r):
            hist[pl.ds(r * LANES, LANES)] = jnp.zeros((LANES,), jnp.float32)

        ones = jnp.ones((LANES,), jnp.float32)

        @pl.loop(0, ch * K // LANES)
        def _h(i):
            d = lax.shift_right_logical(pslab[pl.ds(i * LANES, LANES)], _SHIFT)
            plsc.addupdate_scatter(hist, [d], ones)

        pltpu.sync_copy(hist, out_hbm.at[c, s])

    return deg_k(flat)


def _edge_pass(table, packed4, npad):
    ch = packed4.shape[2]
    h = table.shape[1]
    rps = npad // NS

    @functools.partial(
        pl.kernel,
        out_type=jax.ShapeDtypeStruct((NC, npad, h), jnp.float32),
        mesh=_MESH,
        scratch_types=[
            pltpu.VMEM((ch, K), jnp.int32),
            pltpu.VMEM((K, h), jnp.float32),
            pltpu.VMEM((K, h), jnp.float32),
            pltpu.VMEM((K,), jnp.int32),
            pltpu.VMEM((K,), jnp.int32),
            pltpu.VMEM((K,), jnp.int32),
            pltpu.VMEM((K,), jnp.int32),
            pltpu.VMEM_SHARED((npad, h), jnp.float32),
            pltpu.SemaphoreType.DMA,
            pltpu.SemaphoreType.DMA,
        ],
    )
    def edge_k(table_hbm, p_hbm, out_hbm, pslab, gba, gbb, sa, sb, da, db,
               acc, gsa, gsb):
        c = lax.axis_index("c")
        s = lax.axis_index("s")
        pltpu.sync_copy(p_hbm.at[c, s], pslab)
        _fill(gba, K, 0.0)
        _zero_acc(gba, acc, s, rps)
        plsc.subcore_barrier()

        _unpack(pslab, 0, sa, da)
        pltpu.async_copy(table_hbm.at[sa], gba, gsa)

        @pl.loop(0, ch, step=2)
        def _go(j0):
            _unpack(pslab, j0 + 1, sb, db)
            pltpu.async_copy(table_hbm.at[sb], gbb, gsb)
            pltpu.make_async_copy(table_hbm.at[sa], gba, gsa).wait()
            pltpu.sync_copy(gba, acc.at[da], add=True)
            @pl.when(j0 + 2 < ch)
            def _next():
                _unpack(pslab, j0 + 2, sa, da)
                pltpu.async_copy(table_hbm.at[sa], gba, gsa)
            pltpu.make_async_copy(table_hbm.at[sb], gbb, gsb).wait()
            pltpu.sync_copy(gbb, acc.at[db], add=True)

        plsc.subcore_barrier()
        _acc_to_hbm(acc, gba, out_hbm, c, s, rps)

    return edge_k(table, packed4)


def _dinv_of(deg_ref, n):
    nw = deg_ref.shape[0]
    cnt = lax.dot_general(
        deg_ref[...], jnp.ones((nw, 1), jnp.float32),
        (((0,), (0,)), ((), ())),
        preferred_element_type=jnp.float32, precision=lax.Precision.HIGHEST,
    )
    return lax.rsqrt(cnt[:n] + 1.0)


def _mm(x, w):
    n = x.shape[0]

    def body(x_ref, w_ref, o_ref):
        o_ref[...] = jnp.dot(
            x_ref[...], w_ref[...],
            preferred_element_type=jnp.float32, precision=lax.Precision.HIGHEST,
        )

    return pl.pallas_call(
        body, out_shape=jax.ShapeDtypeStruct((n, w.shape[1]), jnp.float32)
    )(x, w)


def _scale(hmat, deg):
    n = hmat.shape[0]

    def body(h_ref, deg_ref, o_ref):
        o_ref[...] = h_ref[...] * _dinv_of(deg_ref, n)

    return pl.pallas_call(
        body, out_shape=jax.ShapeDtypeStruct(hmat.shape, jnp.float32)
    )(hmat, deg)


def _mid(accp, hp, deg, b, g, bt, w2):
    n = hp.shape[0]

    def body(acc_ref, h_ref, deg_ref, b_ref, g_ref, bt_ref, w2_ref, o_ref):
        dinv = _dinv_of(deg_ref, n)
        t = (acc_ref[0, :n, :] + acc_ref[1, :n, :] + h_ref[...]) * dinv + b_ref[...]
        mu = jnp.mean(t, axis=0, keepdims=True)
        xc = t - mu
        var = jnp.mean(xc * xc, axis=0, keepdims=True)
        z = xc * lax.rsqrt(var + 1e-5) * g_ref[...] + bt_ref[...]
        z = jnp.maximum(z, 0.0)
        o_ref[...] = jnp.dot(
            z, w2_ref[...],
            preferred_element_type=jnp.float32, precision=lax.Precision.HIGHEST,
        ) * dinv

    return pl.pallas_call(
        body, out_shape=jax.ShapeDtypeStruct((n, w2.shape[1]), jnp.float32)
    )(accp, hp, deg, b, g, bt, w2)


def _fin(accp, hp, deg, b, g, bt, wc, bc):
    n = hp.shape[0]

    def body(acc_ref, h_ref, deg_ref, b_ref, g_ref, bt_ref, wc_ref, bc_ref, o_ref):
        dinv = _dinv_of(deg_ref, n)
        t = (acc_ref[0, :n, :] + acc_ref[1, :n, :] + h_ref[...]) * dinv + b_ref[...]
        mu = jnp.mean(t, axis=0, keepdims=True)
        xc = t - mu
        var = jnp.mean(xc * xc, axis=0, keepdims=True)
        z = xc * lax.rsqrt(var + 1e-5) * g_ref[...] + bt_ref[...]
        z = jnp.maximum(z, 0.0)
        o_ref[...] = jnp.dot(
            z, wc_ref[...],
            preferred_element_type=jnp.float32, precision=lax.Precision.HIGHEST,
        ) + bc_ref[...]

    return pl.pallas_call(
        body, out_shape=jax.ShapeDtypeStruct((n, wc.shape[1]), jnp.float32)
    )(accp, hp, deg, b, g, bt, wc, bc)


def kernel(x, edge_index, W1, b1, g1, bt1, W2, b2, g2, bt2, Wc, bc):
    n = x.shape[0]
    e = edge_index.shape[1]
    npad = -(-(n + 1) // (NS * 8)) * (NS * 8)
    assert npad <= (1 << (31 - _SHIFT)) and n <= _MASK
    ch = -(-e // (NC * NS * K))
    ch += ch % 2
    pad = NC * NS * ch * K - e
    ar = jnp.arange(pad, dtype=edge_index.dtype)
    src = jnp.concatenate([edge_index[0], ar % 8])
    dst = jnp.concatenate([edge_index[1], n + ar % (npad - n)])
    packed4 = (src | (dst << _SHIFT)).reshape(NC, NS, ch, K)

    mm1 = _mm(x, W1)
    deg = _deg_pass(packed4, npad).reshape(NC * NS, npad)
    h1 = _scale(mm1, deg)
    acc1 = _edge_pass(h1, packed4, npad)
    h2 = _mid(
        acc1, h1, deg,
        b1.reshape(1, -1), g1.reshape(1, -1), bt1.reshape(1, -1), W2,
    )
    acc2 = _edge_pass(h2, packed4, npad)
    return _fin(
        acc2, h2, deg,
        b2.reshape(1, -1), g2.reshape(1, -1), bt2.reshape(1, -1), Wc,
        bc.reshape(1, -1),
    )
